# baseline (speedup 1.0000x reference)
import functools

import jax
import jax.numpy as jnp
from jax import lax
from jax.experimental import pallas as pl
from jax.experimental.pallas import tpu as pltpu
from jax.experimental.pallas import tpu_sc as plsc

NUM_CORES = 2
NUM_SUBCORES = 16
NW = NUM_CORES * NUM_SUBCORES

D = 64
IDX_W = 128
K = 4
CHUNK = K * IDX_W


def _gather_kernel(idx_hbm, table_hbm, out_hbm, idx_v, rows_v, sem):
    n_chunks_total = idx_hbm.shape[0]
    n_chunks = n_chunks_total // NW
    wid = lax.axis_index("s") * NUM_CORES + lax.axis_index("c")

    def body(g, _):
        chunk = wid * n_chunks + g
        pltpu.sync_copy(idx_hbm.at[chunk], idx_v)
        copies = [
            pltpu.async_copy(
                table_hbm.at[idx_v.at[j]],
                rows_v.at[pl.ds(j * IDX_W, IDX_W)],
                sem,
            )
            for j in range(K)
        ]
        for c in copies:
            c.wait()
        pltpu.sync_copy(rows_v, out_hbm.at[pl.ds(chunk * CHUNK, CHUNK)])
        return _

    lax.fori_loop(0, n_chunks, body, None)


def kernel(labels, label_embed_weight):
    B0, B1 = labels.shape
    B = B0 * B1
    n_chunks_total = B // CHUNK
    assert B % (CHUNK * NW) == 0

    idx = labels.reshape(n_chunks_total, K, IDX_W).astype(jnp.int32)

    run = pl.kernel(
        _gather_kernel,
        out_type=jax.ShapeDtypeStruct((B, D), jnp.float32),
        mesh=plsc.VectorSubcoreMesh(
            core_axis_name="c", subcore_axis_name="s",
            num_cores=NUM_CORES, num_subcores=NUM_SUBCORES,
        ),
        scratch_types=[
            pltpu.VMEM((K, IDX_W), jnp.int32),
            pltpu.VMEM((CHUNK, D), jnp.float32),
            pltpu.SemaphoreType.DMA,
        ],
        compiler_params=pltpu.CompilerParams(use_tc_tiling_on_sc=False),
    )
    out = run(idx, label_embed_weight)
    return out.reshape(B0, B1, D)


# speedup vs baseline: 1.0299x; 1.0299x over previous
import jax
import jax.numpy as jnp
from jax import lax
from jax.experimental import pallas as pl
from jax.experimental.pallas import tpu as pltpu
from jax.experimental.pallas import tpu_sc as plsc

NUM_CORES = 2
NUM_SUBCORES = 16
NW = NUM_CORES * NUM_SUBCORES

D = 64
IDX_W = 128
K = 4
CHUNK = K * IDX_W
NB = 3


def _gather_kernel(idx_hbm, table_hbm, out_hbm,
                   idx_v, rows0, rows1, rows2,
                   sg0, sg1, sg2, so0, so1, so2):
    rows = (rows0, rows1, rows2)
    sg = (sg0, sg1, sg2)
    so = (so0, so1, so2)
    cpw = idx_hbm.shape[1] // K
    wid = lax.axis_index("s") * NUM_CORES + lax.axis_index("c")
    out_base = wid * cpw * CHUNK

    def start_gather(c, b):
        for j in range(K):
            pltpu.async_copy(
                table_hbm.at[idx_v.at[c * K + j]],
                rows[b].at[pl.ds(j * IDX_W, IDX_W)],
                sg[b],
            )

    def wait_gather(c, b):
        for j in range(K):
            pltpu.make_async_copy(
                table_hbm.at[idx_v.at[c * K + j]],
                rows[b].at[pl.ds(j * IDX_W, IDX_W)],
                sg[b],
            ).wait()

    def start_out(c, b):
        pltpu.async_copy(
            rows[b], out_hbm.at[pl.ds(out_base + c * CHUNK, CHUNK)], so[b])

    def wait_out(b):
        pltpu.make_async_copy(
            rows[b], out_hbm.at[pl.ds(out_base, CHUNK)], so[b]).wait()

    pltpu.sync_copy(idx_hbm.at[wid], idx_v)

    start_gather(0, 0)
    start_gather(1, 1)
    wait_gather(0, 0); start_out(0, 0); start_gather(2, 2)
    wait_gather(1, 1); start_out(1, 1); wait_out(0); start_gather(3, 0)
    wait_gather(2, 2); start_out(2, 2); wait_out(1); start_gather(4, 1)

    def body(i, _):
        for b in range(NB):
            c = NB * i + b
            wait_gather(c, b)
            start_out(c, b)
            wait_out((b + 2) % NB)
            start_gather(c + 2, (b + 2) % NB)
        return _

    lax.fori_loop(1, (cpw - 2) // NB, body, None)

    c0, c1 = cpw - 2, cpw - 1
    b0, b1 = c0 % NB, c1 % NB
    wait_gather(c0, b0); start_out(c0, b0)
    wait_gather(c1, b1); start_out(c1, b1)
    wait_out((b1 + 1) % NB); wait_out(b0); wait_out(b1)


def kernel(labels, label_embed_weight):
    B0, B1 = labels.shape
    B = B0 * B1
    cpw = B // (NW * CHUNK)
    assert B == NW * cpw * CHUNK and cpw % NB == 2 % NB

    idx = labels.reshape(NW, cpw * K, IDX_W).astype(jnp.int32)

    run = pl.kernel(
        _gather_kernel,
        out_type=jax.ShapeDtypeStruct((B, D), jnp.float32),
        mesh=plsc.VectorSubcoreMesh(
            core_axis_name="c", subcore_axis_name="s",
            num_cores=NUM_CORES, num_subcores=NUM_SUBCORES,
        ),
        scratch_types=[
            pltpu.VMEM((cpw * K, IDX_W), jnp.int32),
            pltpu.VMEM((CHUNK, D), jnp.float32),
            pltpu.VMEM((CHUNK, D), jnp.float32),
            pltpu.VMEM((CHUNK, D), jnp.float32),
            pltpu.SemaphoreType.DMA,
            pltpu.SemaphoreType.DMA,
            pltpu.SemaphoreType.DMA,
            pltpu.SemaphoreType.DMA,
            pltpu.SemaphoreType.DMA,
            pltpu.SemaphoreType.DMA,
        ],
        compiler_params=pltpu.CompilerParams(use_tc_tiling_on_sc=False),
    )
    out = run(idx, label_embed_weight)
    return out.reshape(B0, B1, D)
